# baseline (speedup 1.0000x reference)
import functools

import jax
import jax.numpy as jnp
from jax import lax
from jax.experimental import pallas as pl
from jax.experimental.pallas import tpu as pltpu
from jax.experimental.pallas import tpu_sc as plsc

N = 10000
E = 160000
IN_C = 256
HID = 128
OUT_C = 3
OUT_P = 16

NC = 2
NS = 16
CHUNK = 125
ROWS_PER_TILE = E // (NC * NS) // CHUNK
NP = 10240
NODES_PER_TILE = NP // NS
ZROWS = 128

_mesh = plsc.VectorSubcoreMesh(core_axis_name="c", subcore_axis_name="s")


DEG_W = 16


def _deg_body(dst2_hbm, out_hbm, dstv, ones_v, zb_v, acc):
    c = lax.axis_index("c")
    s = lax.axis_index("s")

    def zb(i, carry):
        zb_v[i, pl.ds(0, 16)] = jnp.zeros((16,), jnp.float32)
        ones_v[i, pl.ds(0, 16)] = jnp.ones((16,), jnp.float32)
        return carry

    lax.fori_loop(0, ZROWS, zb, 0)
    for k in range(NODES_PER_TILE // ZROWS):
        pltpu.sync_copy(zb_v, acc.at[pl.ds(s * NODES_PER_TILE + k * ZROWS, ZROWS)])
    plsc.subcore_barrier()

    base = (c * NS + s) * ROWS_PER_TILE
    pltpu.sync_copy(dst2_hbm.at[pl.ds(base, ROWS_PER_TILE)], dstv)

    def body(j, carry):
        pltpu.sync_copy(ones_v.at[pl.ds(0, CHUNK)], acc.at[dstv.at[j]], add=True)
        return carry

    lax.fori_loop(0, ROWS_PER_TILE, body, 0)
    plsc.subcore_barrier()
    sl = pl.ds(s * NODES_PER_TILE, NODES_PER_TILE)
    pltpu.sync_copy(acc.at[sl], out_hbm.at[c, sl])


_deg_kernel = pl.kernel(
    _deg_body,
    out_type=jax.ShapeDtypeStruct((NC, NP, DEG_W), jnp.float32),
    mesh=_mesh,
    scratch_types=[
        pltpu.VMEM((ROWS_PER_TILE, CHUNK), jnp.int32),
        pltpu.VMEM((ZROWS, DEG_W), jnp.float32),
        pltpu.VMEM((ZROWS, DEG_W), jnp.float32),
        pltpu.VMEM_SHARED((NP, DEG_W), jnp.float32),
    ],
    compiler_params=pltpu.CompilerParams(use_tc_tiling_on_sc=False),
)


def _agg_body(hs_hbm, src2_hbm, dst2_hbm, out_hbm, srcv, dstv, rows, *rest, W, group):
    gsems = rest[:group]
    ssems = rest[group : 2 * group]
    acc = rest[2 * group]
    c = lax.axis_index("c")
    s = lax.axis_index("s")

    @pl.when(c == 0)
    def _():
        pltpu.sync_copy(
            hs_hbm.at[pl.ds(s * 624, 624)], acc.at[pl.ds(s * 624, 624)]
        )

        @pl.when(s == 0)
        def _():
            pltpu.sync_copy(
                hs_hbm.at[pl.ds(9984, 16)], acc.at[pl.ds(9984, 16)]
            )

    def zb(i, carry):
        for k in range(W // 16):
            rows[0, i, pl.ds(k * 16, 16)] = jnp.zeros((16,), jnp.float32)
        return carry

    lax.fori_loop(0, ZROWS, zb, 0)

    @pl.when(c == 1)
    def _():
        for k in range(NODES_PER_TILE // ZROWS):
            pltpu.sync_copy(
                rows.at[0], acc.at[pl.ds(s * NODES_PER_TILE + k * ZROWS, ZROWS)]
            )

    plsc.subcore_barrier()

    base = (c * NS + s) * ROWS_PER_TILE
    pltpu.sync_copy(src2_hbm.at[pl.ds(base, ROWS_PER_TILE)], srcv)
    pltpu.sync_copy(dst2_hbm.at[pl.ds(base, ROWS_PER_TILE)], dstv)

    if group == 2:
        r0 = rows.at[0, pl.ds(0, CHUNK)]
        r1 = rows.at[1, pl.ds(0, CHUNK)]
        pltpu.async_copy(hs_hbm.at[srcv.at[0]], r0, gsems[0]).wait()
        npair = ROWS_PER_TILE // 2

        def pair(i, carry):
            j0 = 2 * i
            g1 = pltpu.async_copy(hs_hbm.at[srcv.at[j0 + 1]], r1, gsems[1])
            s0 = pltpu.async_copy(r0, acc.at[dstv.at[j0]], ssems[0], add=True)
            g1.wait()
            s0.wait()

            @pl.when(i < npair - 1)
            def _():
                g2 = pltpu.async_copy(hs_hbm.at[srcv.at[j0 + 2]], r0, gsems[0])
                s1 = pltpu.async_copy(r1, acc.at[dstv.at[j0 + 1]], ssems[1], add=True)
                g2.wait()
                s1.wait()

            @pl.when(i == npair - 1)
            def _():
                pltpu.async_copy(r1, acc.at[dstv.at[j0 + 1]], ssems[1], add=True).wait()

            return carry

        lax.fori_loop(0, npair, pair, 0)
    else:
        def body(i, carry):
            j = group * i
            gd = []
            for p in range(group):
                gd.append(
                    pltpu.async_copy(
                        hs_hbm.at[srcv.at[j + p]],
                        rows.at[p, pl.ds(0, CHUNK)],
                        gsems[p],
                    )
                )
            sd = []
            for p in range(group):
                gd[p].wait()
                sd.append(
                    pltpu.async_copy(
                        rows.at[p, pl.ds(0, CHUNK)],
                        acc.at[dstv.at[j + p]],
                        ssems[p],
                        add=True,
                    )
                )
            for p in range(group):
                sd[p].wait()
            return carry

        lax.fori_loop(0, ROWS_PER_TILE // group, body, 0)
    plsc.subcore_barrier()
    pltpu.sync_copy(
        acc.at[pl.ds(s * NODES_PER_TILE, NODES_PER_TILE)],
        out_hbm.at[c, pl.ds(s * NODES_PER_TILE, NODES_PER_TILE)],
    )


def _make_agg(W, group):
    return pl.kernel(
        functools.partial(_agg_body, W=W, group=group),
        out_type=jax.ShapeDtypeStruct((NC, NP, W), jnp.float32),
        mesh=_mesh,
        scratch_types=(
            [
                pltpu.VMEM((ROWS_PER_TILE, CHUNK), jnp.int32),
                pltpu.VMEM((ROWS_PER_TILE, CHUNK), jnp.int32),
                pltpu.VMEM((group, ZROWS, W), jnp.float32),
            ]
            + [pltpu.SemaphoreType.DMA] * (2 * group)
            + [pltpu.VMEM_SHARED((NP, W), jnp.float32)]
        ),
        compiler_params=pltpu.CompilerParams(
            use_tc_tiling_on_sc=None if W >= 128 else False
        ),
    )


_agg128 = _make_agg(HID, 2)
_agg16 = _make_agg(OUT_P, 8)


_BR = 2000
_GRID = N // _BR


def _mm1_body(x_ref, w_ref, h_ref):
    h_ref[...] = jnp.dot(x_ref[...], w_ref[...], preferred_element_type=jnp.float32)


_mm1 = pl.pallas_call(
    _mm1_body,
    grid=(_GRID,),
    in_specs=[
        pl.BlockSpec((_BR, IN_C), lambda i: (i, 0)),
        pl.BlockSpec((IN_C, HID), lambda i: (0, 0)),
    ],
    out_specs=pl.BlockSpec((_BR, HID), lambda i: (i, 0)),
    out_shape=jax.ShapeDtypeStruct((N, HID), jnp.float32),
)


def _scale1_body(dp_ref, h_ref, hs_ref, d_ref):
    d = lax.rsqrt(dp_ref[0, :, :1] + dp_ref[1, :, :1] + 1.0)
    hs_ref[...] = h_ref[...] * d
    d_ref[...] = d


_scale1 = pl.pallas_call(
    _scale1_body,
    grid=(_GRID,),
    in_specs=[
        pl.BlockSpec((NC, _BR, DEG_W), lambda i: (0, i, 0)),
        pl.BlockSpec((_BR, HID), lambda i: (i, 0)),
    ],
    out_specs=[
        pl.BlockSpec((_BR, HID), lambda i: (i, 0)),
        pl.BlockSpec((_BR, 1), lambda i: (i, 0)),
    ],
    out_shape=[
        jax.ShapeDtypeStruct((N, HID), jnp.float32),
        jax.ShapeDtypeStruct((N, 1), jnp.float32),
    ],
)


def _lin2_body(agg_ref, d_ref, b1_ref, w2_ref, out_ref):
    x1 = jnp.maximum(
        d_ref[...] * (agg_ref[0] + agg_ref[1]) + b1_ref[...], 0.0
    )
    out_ref[...] = d_ref[...] * jnp.dot(
        x1, w2_ref[...], preferred_element_type=jnp.float32
    )


_lin2 = pl.pallas_call(
    _lin2_body,
    grid=(_GRID,),
    in_specs=[
        pl.BlockSpec((NC, _BR, HID), lambda i: (0, i, 0)),
        pl.BlockSpec((_BR, 1), lambda i: (i, 0)),
        pl.BlockSpec((1, HID), lambda i: (0, 0)),
        pl.BlockSpec((HID, OUT_P), lambda i: (0, 0)),
    ],
    out_specs=pl.BlockSpec((_BR, OUT_P), lambda i: (i, 0)),
    out_shape=jax.ShapeDtypeStruct((N, OUT_P), jnp.float32),
)


def _fin_body(e_ref, d_ref, b2_ref, out_ref):
    v = d_ref[...] * (e_ref[0] + e_ref[1])
    out_ref[...] = v[:, : OUT_C] + b2_ref[...]


_FBR = 5000

_fin = pl.pallas_call(
    _fin_body,
    grid=(N // _FBR,),
    in_specs=[
        pl.BlockSpec((NC, _FBR, OUT_P), lambda i: (0, i, 0)),
        pl.BlockSpec((_FBR, 1), lambda i: (i, 0)),
        pl.BlockSpec((1, OUT_C), lambda i: (0, 0)),
    ],
    out_specs=pl.BlockSpec((_FBR, OUT_C), lambda i: (i, 0)),
    out_shape=jax.ShapeDtypeStruct((N, OUT_C), jnp.float32),
)


def kernel(features, edges, edges2, edge_features, additional_feature, W1, b1, W2, b2):
    src2 = edges[0].reshape(E // CHUNK, CHUNK)
    dst2 = edges[1].reshape(E // CHUNK, CHUNK)

    degp = _deg_kernel(dst2)
    h = _mm1(features, W1)
    hs, d = _scale1(degp, h)
    agg = _agg128(hs, src2, dst2)

    w2p = jnp.pad(W2, ((0, 0), (0, OUT_P - W2.shape[1])))
    b1r = b1.reshape(1, HID)
    h2p = _lin2(agg, d, b1r, w2p)

    agg2 = _agg16(h2p, src2, dst2)
    return _fin(agg2, d, b2.reshape(1, OUT_C))


# speedup vs baseline: 1.0145x; 1.0145x over previous
import functools

import jax
import jax.numpy as jnp
from jax import lax
from jax.experimental import pallas as pl
from jax.experimental.pallas import tpu as pltpu
from jax.experimental.pallas import tpu_sc as plsc

N = 10000
E = 160000
IN_C = 256
HID = 128
OUT_C = 3
OUT_P = 16

NC = 2
NS = 16
CHUNK = 125
ROWS_PER_TILE = E // (NC * NS) // CHUNK
NP = 10240
NODES_PER_TILE = NP // NS
ZROWS = 128

_mesh = plsc.VectorSubcoreMesh(core_axis_name="c", subcore_axis_name="s")


DEG_W = 16


def _deg_body(dst2_hbm, out_hbm, dstv, ones_v, zb_v, acc):
    c = lax.axis_index("c")
    s = lax.axis_index("s")

    def zb(i, carry):
        zb_v[i, pl.ds(0, 16)] = jnp.zeros((16,), jnp.float32)
        ones_v[i, pl.ds(0, 16)] = jnp.ones((16,), jnp.float32)
        return carry

    lax.fori_loop(0, ZROWS, zb, 0)
    for k in range(NODES_PER_TILE // ZROWS):
        pltpu.sync_copy(zb_v, acc.at[pl.ds(s * NODES_PER_TILE + k * ZROWS, ZROWS)])
    plsc.subcore_barrier()

    base = (c * NS + s) * ROWS_PER_TILE
    pltpu.sync_copy(dst2_hbm.at[pl.ds(base, ROWS_PER_TILE)], dstv)

    def body(j, carry):
        pltpu.sync_copy(ones_v.at[pl.ds(0, CHUNK)], acc.at[dstv.at[j]], add=True)
        return carry

    lax.fori_loop(0, ROWS_PER_TILE, body, 0)
    plsc.subcore_barrier()
    sl = pl.ds(s * NODES_PER_TILE, NODES_PER_TILE)
    pltpu.sync_copy(acc.at[sl], out_hbm.at[c, sl])


_deg_kernel = pl.kernel(
    _deg_body,
    out_type=jax.ShapeDtypeStruct((NC, NP, DEG_W), jnp.float32),
    mesh=_mesh,
    scratch_types=[
        pltpu.VMEM((ROWS_PER_TILE, CHUNK), jnp.int32),
        pltpu.VMEM((ZROWS, DEG_W), jnp.float32),
        pltpu.VMEM((ZROWS, DEG_W), jnp.float32),
        pltpu.VMEM_SHARED((NP, DEG_W), jnp.float32),
    ],
    compiler_params=pltpu.CompilerParams(use_tc_tiling_on_sc=False),
)


def _agg_body(hs_hbm, src2_hbm, dst2_hbm, out_hbm, srcv, dstv, rows, *rest, W, group):
    gsems = rest[:group]
    ssems = rest[group : 2 * group]
    acc = rest[2 * group]
    c = lax.axis_index("c")
    s = lax.axis_index("s")

    def zb(i, carry):
        for k in range(W // 16):
            rows[0, i, pl.ds(k * 16, 16)] = jnp.zeros((16,), jnp.float32)
        return carry

    lax.fori_loop(0, ZROWS, zb, 0)
    for k in range(NODES_PER_TILE // ZROWS):
        pltpu.sync_copy(
            rows.at[0], acc.at[pl.ds(s * NODES_PER_TILE + k * ZROWS, ZROWS)]
        )
    plsc.subcore_barrier()

    base = (c * NS + s) * ROWS_PER_TILE
    pltpu.sync_copy(src2_hbm.at[pl.ds(base, ROWS_PER_TILE)], srcv)
    pltpu.sync_copy(dst2_hbm.at[pl.ds(base, ROWS_PER_TILE)], dstv)

    if group == 2:
        r0 = rows.at[0, pl.ds(0, CHUNK)]
        r1 = rows.at[1, pl.ds(0, CHUNK)]
        pltpu.async_copy(hs_hbm.at[srcv.at[0]], r0, gsems[0]).wait()
        npair = ROWS_PER_TILE // 2

        def pair(i, carry):
            j0 = 2 * i
            g1 = pltpu.async_copy(hs_hbm.at[srcv.at[j0 + 1]], r1, gsems[1])
            s0 = pltpu.async_copy(r0, acc.at[dstv.at[j0]], ssems[0], add=True)
            g1.wait()
            s0.wait()

            @pl.when(i < npair - 1)
            def _():
                g2 = pltpu.async_copy(hs_hbm.at[srcv.at[j0 + 2]], r0, gsems[0])
                s1 = pltpu.async_copy(r1, acc.at[dstv.at[j0 + 1]], ssems[1], add=True)
                g2.wait()
                s1.wait()

            @pl.when(i == npair - 1)
            def _():
                pltpu.async_copy(r1, acc.at[dstv.at[j0 + 1]], ssems[1], add=True).wait()

            return carry

        lax.fori_loop(0, npair, pair, 0)
    else:
        def body(i, carry):
            j = group * i
            gd = []
            for p in range(group):
                gd.append(
                    pltpu.async_copy(
                        hs_hbm.at[srcv.at[j + p]],
                        rows.at[p, pl.ds(0, CHUNK)],
                        gsems[p],
                    )
                )
            sd = []
            for p in range(group):
                gd[p].wait()
                sd.append(
                    pltpu.async_copy(
                        rows.at[p, pl.ds(0, CHUNK)],
                        acc.at[dstv.at[j + p]],
                        ssems[p],
                        add=True,
                    )
                )
            for p in range(group):
                sd[p].wait()
            return carry

        lax.fori_loop(0, ROWS_PER_TILE // group, body, 0)
    plsc.subcore_barrier()
    pltpu.sync_copy(
        acc.at[pl.ds(s * NODES_PER_TILE, NODES_PER_TILE)],
        out_hbm.at[c, pl.ds(s * NODES_PER_TILE, NODES_PER_TILE)],
    )


def _make_agg(W, group):
    return pl.kernel(
        functools.partial(_agg_body, W=W, group=group),
        out_type=jax.ShapeDtypeStruct((NC, NP, W), jnp.float32),
        mesh=_mesh,
        scratch_types=(
            [
                pltpu.VMEM((ROWS_PER_TILE, CHUNK), jnp.int32),
                pltpu.VMEM((ROWS_PER_TILE, CHUNK), jnp.int32),
                pltpu.VMEM((group, ZROWS, W), jnp.float32),
            ]
            + [pltpu.SemaphoreType.DMA] * (2 * group)
            + [pltpu.VMEM_SHARED((NP, W), jnp.float32)]
        ),
        compiler_params=pltpu.CompilerParams(
            use_tc_tiling_on_sc=None if W >= 128 else False
        ),
    )


_agg128 = _make_agg(HID, 2)
_agg16 = _make_agg(OUT_P, 8)


_BR = 2000
_GRID = N // _BR


def _mm1_body(x_ref, w_ref, h_ref):
    h_ref[...] = jnp.dot(x_ref[...], w_ref[...], preferred_element_type=jnp.float32)


_mm1 = pl.pallas_call(
    _mm1_body,
    grid=(_GRID,),
    in_specs=[
        pl.BlockSpec((_BR, IN_C), lambda i: (i, 0)),
        pl.BlockSpec((IN_C, HID), lambda i: (0, 0)),
    ],
    out_specs=pl.BlockSpec((_BR, HID), lambda i: (i, 0)),
    out_shape=jax.ShapeDtypeStruct((N, HID), jnp.float32),
)


def _scale1_body(dp_ref, h_ref, hs_ref, d_ref):
    d = lax.rsqrt(dp_ref[0, :, :1] + dp_ref[1, :, :1] + 1.0)
    hs_ref[...] = h_ref[...] * d
    d_ref[...] = d


_scale1 = pl.pallas_call(
    _scale1_body,
    grid=(_GRID,),
    in_specs=[
        pl.BlockSpec((NC, _BR, DEG_W), lambda i: (0, i, 0)),
        pl.BlockSpec((_BR, HID), lambda i: (i, 0)),
    ],
    out_specs=[
        pl.BlockSpec((_BR, HID), lambda i: (i, 0)),
        pl.BlockSpec((_BR, 1), lambda i: (i, 0)),
    ],
    out_shape=[
        jax.ShapeDtypeStruct((N, HID), jnp.float32),
        jax.ShapeDtypeStruct((N, 1), jnp.float32),
    ],
)


def _lin2_body(agg_ref, hs_ref, d_ref, b1_ref, w2_ref, out_ref):
    x1 = jnp.maximum(
        d_ref[...] * (agg_ref[0] + agg_ref[1] + hs_ref[...]) + b1_ref[...], 0.0
    )
    out_ref[...] = d_ref[...] * jnp.dot(
        x1, w2_ref[...], preferred_element_type=jnp.float32
    )


_lin2 = pl.pallas_call(
    _lin2_body,
    grid=(_GRID,),
    in_specs=[
        pl.BlockSpec((NC, _BR, HID), lambda i: (0, i, 0)),
        pl.BlockSpec((_BR, HID), lambda i: (i, 0)),
        pl.BlockSpec((_BR, 1), lambda i: (i, 0)),
        pl.BlockSpec((1, HID), lambda i: (0, 0)),
        pl.BlockSpec((HID, OUT_P), lambda i: (0, 0)),
    ],
    out_specs=pl.BlockSpec((_BR, OUT_P), lambda i: (i, 0)),
    out_shape=jax.ShapeDtypeStruct((N, OUT_P), jnp.float32),
)


def _fin_body(e_ref, h2_ref, d_ref, b2_ref, out_ref):
    v = d_ref[...] * (e_ref[0] + e_ref[1] + h2_ref[...])
    out_ref[...] = v[:, : OUT_C] + b2_ref[...]


_FBR = 5000

_fin = pl.pallas_call(
    _fin_body,
    grid=(N // _FBR,),
    in_specs=[
        pl.BlockSpec((NC, _FBR, OUT_P), lambda i: (0, i, 0)),
        pl.BlockSpec((_FBR, OUT_P), lambda i: (i, 0)),
        pl.BlockSpec((_FBR, 1), lambda i: (i, 0)),
        pl.BlockSpec((1, OUT_C), lambda i: (0, 0)),
    ],
    out_specs=pl.BlockSpec((_FBR, OUT_C), lambda i: (i, 0)),
    out_shape=jax.ShapeDtypeStruct((N, OUT_C), jnp.float32),
)


def kernel(features, edges, edges2, edge_features, additional_feature, W1, b1, W2, b2):
    src2 = edges[0].reshape(E // CHUNK, CHUNK)
    dst2 = edges[1].reshape(E // CHUNK, CHUNK)

    degp = _deg_kernel(dst2)
    h = _mm1(features, W1)
    hs, d = _scale1(degp, h)
    agg = _agg128(hs, src2, dst2)

    w2p = jnp.pad(W2, ((0, 0), (0, OUT_P - W2.shape[1])))
    b1r = b1.reshape(1, HID)
    h2p = _lin2(agg, hs, d, b1r, w2p)

    agg2 = _agg16(h2p, src2, dst2)
    return _fin(agg2, h2p, d, b2.reshape(1, OUT_C))
